# baseline (speedup 1.0000x reference)
import functools

import jax
import jax.numpy as jnp
from jax import lax
from jax.experimental import pallas as pl
from jax.experimental.pallas import tpu as pltpu
from jax.experimental.pallas import tpu_sc as plsc

N = 10000
E = 320000
D = 128
NC = 2
NS = 16
NW = NC * NS
B = 128
RPW = 80
EPAD_ROWS = NW * RPW
EPAD = EPAD_ROWS * B
ACC_ROWS = 10240
RPS = ACC_ROWS // NS
IDXC = 8
RPW0 = 2 * RPW
ROW_BLK = 1000


def _sc_degree(dst2d):
    mesh = plsc.VectorSubcoreMesh(core_axis_name="c", subcore_axis_name="s")

    @functools.partial(
        pl.kernel,
        out_type=jax.ShapeDtypeStruct((NC, ACC_ROWS, 16), jnp.float32),
        mesh=mesh,
        scratch_types=[
            pltpu.VMEM((RPW, B), jnp.int32),
            pltpu.VMEM((B, 16), jnp.float32),
            pltpu.VMEM((B, 16), jnp.float32),
            pltpu.VMEM_SHARED((ACC_ROWS, 16), jnp.float32),
        ],
    )
    def k(dst_hbm, out_hbm, dst_v, ones_v, zero_v, acc):
        cid = lax.axis_index("c")
        sid = lax.axis_index("s")
        wid = sid * NC + cid

        pltpu.sync_copy(dst_hbm.at[pl.ds(wid * RPW, RPW)], dst_v)

        @pl.loop(0, B)
        def _(r):
            ones_v[r, pl.ds(0, 16)] = jnp.ones((16,), jnp.float32)
            zero_v[r, pl.ds(0, 16)] = jnp.zeros((16,), jnp.float32)

        @pl.loop(0, RPS, step=B)
        def _(rr):
            pltpu.sync_copy(zero_v, acc.at[pl.ds(sid * RPS + rr, B)])

        plsc.subcore_barrier()

        @pl.loop(0, RPW)
        def _(i):
            pltpu.sync_copy(ones_v, acc.at[dst_v.at[i]], add=True)

        plsc.subcore_barrier()

        @pl.loop(0, RPS, step=B)
        def _(rr):
            pltpu.sync_copy(acc.at[pl.ds(sid * RPS + rr, B)],
                            out_hbm.at[cid, pl.ds(sid * RPS + rr, B)])

    return k(dst2d)


def _sc_aggregate(hn, src2d, dst2d):
    mesh = plsc.VectorSubcoreMesh(core_axis_name="c", subcore_axis_name="s")

    @functools.partial(
        pl.kernel,
        out_type=jax.ShapeDtypeStruct((1, ACC_ROWS, D), jnp.float32),
        mesh=mesh,
        scratch_types=[
            pltpu.VMEM((IDXC, B), jnp.int32),
            pltpu.VMEM((IDXC, B), jnp.int32),
            pltpu.VMEM((B, D), jnp.float32),
            pltpu.VMEM((B, D), jnp.float32),
            pltpu.VMEM_SHARED((ACC_ROWS, D), jnp.float32),
            pltpu.SemaphoreType.DMA,
            pltpu.SemaphoreType.DMA,
        ],
    )
    def k(hn_hbm, src_hbm, dst_hbm, out_hbm,
          src_v, dst_v, bufa, bufb, acc, sema, semb):
        cid = lax.axis_index("c")
        sid = lax.axis_index("s")

        @pl.when(cid == 0)
        def _():
            @pl.loop(0, B)
            def _(r):
                @pl.loop(0, D, step=16)
                def _(c):
                    bufa[r, pl.ds(c, 16)] = jnp.zeros((16,), jnp.float32)

            @pl.loop(0, RPS, step=B)
            def _(rr):
                pltpu.sync_copy(bufa, acc.at[pl.ds(sid * RPS + rr, B)])

            plsc.subcore_barrier()

            @pl.loop(0, RPW0, step=IDXC)
            def _(base):
                row0 = sid * RPW0
                pltpu.sync_copy(src_hbm.at[pl.ds(row0 + base, IDXC)], src_v)
                pltpu.sync_copy(dst_hbm.at[pl.ds(row0 + base, IDXC)], dst_v)

                @pl.loop(0, IDXC, step=2)
                def _(i):
                    cpa = pltpu.async_copy(hn_hbm.at[src_v.at[i]], bufa, sema)
                    cpb = pltpu.async_copy(hn_hbm.at[src_v.at[i + 1]],
                                           bufb, semb)
                    cpa.wait()
                    pltpu.sync_copy(bufa, acc.at[dst_v.at[i]], add=True)
                    cpb.wait()
                    pltpu.sync_copy(bufb, acc.at[dst_v.at[i + 1]], add=True)

            plsc.subcore_barrier()

            @pl.loop(0, RPS, step=B)
            def _(rr):
                pltpu.sync_copy(acc.at[pl.ds(sid * RPS + rr, B)],
                                out_hbm.at[0, pl.ds(sid * RPS + rr, B)])

    return k(hn, src2d, dst2d)


def _tc_matmul_scale(x, w, deg):
    def body(x_ref, w_ref, d_ref, hn_ref, dv_ref):
        d = d_ref[0] + d_ref[1] + 1.0
        dv = lax.rsqrt(d)
        dv_ref[...] = dv
        h = jnp.dot(x_ref[...], w_ref[...],
                    preferred_element_type=jnp.float32)
        hn_ref[...] = h * dv[:, 0:1]

    return pl.pallas_call(
        body,
        grid=(N // ROW_BLK,),
        in_specs=[pl.BlockSpec((ROW_BLK, D), lambda r: (r, 0)),
                  pl.BlockSpec((D, D), lambda r: (0, 0)),
                  pl.BlockSpec((NC, ROW_BLK, 16), lambda r: (0, r, 0))],
        out_specs=[pl.BlockSpec((ROW_BLK, D), lambda r: (r, 0)),
                   pl.BlockSpec((ROW_BLK, 16), lambda r: (r, 0))],
        out_shape=[jax.ShapeDtypeStruct((N, D), jnp.float32),
                   jax.ShapeDtypeStruct((N, 16), jnp.float32)],
    )(x, w, deg)


def _tc_layer(acc, hnp, dv16, b2d, w):
    def body(a_ref, h_ref, d_ref, b_ref, w_ref, o_ref):
        dv = d_ref[:, 0:1]
        s = (a_ref[0] + h_ref[...]) * dv + b_ref[...]
        a = jnp.maximum(s, 0.0)
        o_ref[...] = jnp.dot(a, w_ref[...],
                             preferred_element_type=jnp.float32) * dv

    return pl.pallas_call(
        body,
        grid=(N // ROW_BLK,),
        in_specs=[pl.BlockSpec((1, ROW_BLK, D), lambda r: (0, r, 0)),
                  pl.BlockSpec((ROW_BLK, D), lambda r: (r, 0)),
                  pl.BlockSpec((ROW_BLK, 16), lambda r: (r, 0)),
                  pl.BlockSpec((1, D), lambda r: (0, 0)),
                  pl.BlockSpec((D, D), lambda r: (0, 0))],
        out_specs=pl.BlockSpec((ROW_BLK, D), lambda r: (r, 0)),
        out_shape=jax.ShapeDtypeStruct((N, D), jnp.float32),
    )(acc, hnp, dv16, b2d, w)


def _tc_final(acc, hnp, dv16, b2d, wl, bl2d):
    def body(a_ref, h_ref, d_ref, b_ref, w_ref, bl_ref, o_ref):
        dv = d_ref[:, 0:1]
        s = (a_ref[0] + h_ref[...]) * dv + b_ref[...]
        a = jnp.maximum(s, 0.0)
        logits = jnp.dot(a, w_ref[...],
                         preferred_element_type=jnp.float32) + bl_ref[...]
        m = jnp.max(logits, axis=1, keepdims=True)
        e = jnp.exp(logits - m)
        o_ref[...] = e / jnp.sum(e, axis=1, keepdims=True)

    return pl.pallas_call(
        body,
        grid=(N // ROW_BLK,),
        in_specs=[pl.BlockSpec((1, ROW_BLK, D), lambda r: (0, r, 0)),
                  pl.BlockSpec((ROW_BLK, D), lambda r: (r, 0)),
                  pl.BlockSpec((ROW_BLK, 16), lambda r: (r, 0)),
                  pl.BlockSpec((1, D), lambda r: (0, 0)),
                  pl.BlockSpec((D, D), lambda r: (0, 0)),
                  pl.BlockSpec((1, D), lambda r: (0, 0))],
        out_specs=pl.BlockSpec((ROW_BLK, D), lambda r: (r, 0)),
        out_shape=jax.ShapeDtypeStruct((N, D), jnp.float32),
    )(acc, hnp, dv16, b2d, wl, bl2d)


def kernel(x, adj_matrix, W1, b1, W2, b2, W3, b3, Wl, bl):
    src = adj_matrix[0].astype(jnp.int32)
    dst = adj_matrix[1].astype(jnp.int32)
    pad = EPAD - E
    src2d = jnp.concatenate(
        [src, jnp.zeros((pad,), jnp.int32)]).reshape(EPAD_ROWS, B)
    dst2d = jnp.concatenate(
        [dst, jnp.full((pad,), N, jnp.int32)]).reshape(EPAD_ROWS, B)

    b1_2d = b1.reshape(1, D)
    b2_2d = b2.reshape(1, D)
    b3_2d = b3.reshape(1, D)
    bl_2d = bl.reshape(1, D)

    deg = _sc_degree(dst2d)
    hn1, dv16 = _tc_matmul_scale(x, W1, deg)

    acc1 = _sc_aggregate(hn1, src2d, dst2d)
    hn2 = _tc_layer(acc1, hn1, dv16, b1_2d, W2)
    acc2 = _sc_aggregate(hn2, src2d, dst2d)
    hn3 = _tc_layer(acc2, hn2, dv16, b2_2d, W3)
    acc3 = _sc_aggregate(hn3, src2d, dst2d)
    return _tc_final(acc3, hn3, dv16, b3_2d, Wl, bl_2d)


# speedup vs baseline: 1.7173x; 1.7173x over previous
import functools

import jax
import jax.numpy as jnp
from jax import lax
from jax.experimental import pallas as pl
from jax.experimental.pallas import tpu as pltpu
from jax.experimental.pallas import tpu_sc as plsc

N = 10000
E = 320000
D = 128
NC = 2
NS = 16
NW = NC * NS
B = 128
RPW = 80
EPAD_ROWS = NW * RPW
EPAD = EPAD_ROWS * B
ACC_ROWS = 10240
RPS = ACC_ROWS // NS
IDXC = 8
RPW0 = 152
RPW1 = RPW * 2 - RPW0
ROW_BLK = 1000


def _sc_degree(dst2d):
    mesh = plsc.VectorSubcoreMesh(core_axis_name="c", subcore_axis_name="s")

    @functools.partial(
        pl.kernel,
        out_type=jax.ShapeDtypeStruct((NC, ACC_ROWS, 16), jnp.float32),
        mesh=mesh,
        scratch_types=[
            pltpu.VMEM((RPW, B), jnp.int32),
            pltpu.VMEM((B, 16), jnp.float32),
            pltpu.VMEM((B, 16), jnp.float32),
            pltpu.VMEM_SHARED((ACC_ROWS, 16), jnp.float32),
        ],
    )
    def k(dst_hbm, out_hbm, dst_v, ones_v, zero_v, acc):
        cid = lax.axis_index("c")
        sid = lax.axis_index("s")
        wid = sid * NC + cid

        pltpu.sync_copy(dst_hbm.at[pl.ds(wid * RPW, RPW)], dst_v)

        @pl.loop(0, B)
        def _(r):
            ones_v[r, pl.ds(0, 16)] = jnp.ones((16,), jnp.float32)
            zero_v[r, pl.ds(0, 16)] = jnp.zeros((16,), jnp.float32)

        @pl.loop(0, RPS, step=B)
        def _(rr):
            pltpu.sync_copy(zero_v, acc.at[pl.ds(sid * RPS + rr, B)])

        plsc.subcore_barrier()

        @pl.loop(0, RPW)
        def _(i):
            pltpu.sync_copy(ones_v, acc.at[dst_v.at[i]], add=True)

        plsc.subcore_barrier()

        @pl.loop(0, RPS, step=B)
        def _(rr):
            pltpu.sync_copy(acc.at[pl.ds(sid * RPS + rr, B)],
                            out_hbm.at[cid, pl.ds(sid * RPS + rr, B)])

    return k(dst2d)


def _sc_aggregate(hn, src2d, dst2d):
    mesh = plsc.VectorSubcoreMesh(core_axis_name="c", subcore_axis_name="s")

    @functools.partial(
        pl.kernel,
        out_type=jax.ShapeDtypeStruct((NC, ACC_ROWS, D), jnp.float32),
        mesh=mesh,
        scratch_types=[
            pltpu.VMEM((IDXC, B), jnp.int32),
            pltpu.VMEM((IDXC, B), jnp.int32),
            pltpu.VMEM((B, D), jnp.float32),
            pltpu.VMEM((B, D), jnp.float32),
            pltpu.VMEM_SHARED((ACC_ROWS, D), jnp.float32),
            pltpu.SemaphoreType.DMA,
            pltpu.SemaphoreType.DMA,
        ],
    )
    def k(hn_hbm, src_hbm, dst_hbm, out_hbm,
          src_v, dst_v, bufa, bufb, acc, sema, semb):
        cid = lax.axis_index("c")
        sid = lax.axis_index("s")

        @pl.loop(0, B)
        def _(r):
            @pl.loop(0, D, step=16)
            def _(c):
                bufa[r, pl.ds(c, 16)] = jnp.zeros((16,), jnp.float32)

        @pl.loop(0, RPS, step=B)
        def _(rr):
            pltpu.sync_copy(bufa, acc.at[pl.ds(sid * RPS + rr, B)])

        plsc.subcore_barrier()

        def edge_loop(row0, nrows):
            @pl.loop(0, nrows, step=IDXC)
            def _(base):
                pltpu.sync_copy(src_hbm.at[pl.ds(row0 + base, IDXC)], src_v)
                pltpu.sync_copy(dst_hbm.at[pl.ds(row0 + base, IDXC)], dst_v)

                @pl.loop(0, IDXC, step=2)
                def _(i):
                    cpa = pltpu.async_copy(hn_hbm.at[src_v.at[i]], bufa, sema)
                    cpb = pltpu.async_copy(hn_hbm.at[src_v.at[i + 1]],
                                           bufb, semb)
                    cpa.wait()
                    pltpu.sync_copy(bufa, acc.at[dst_v.at[i]], add=True)
                    cpb.wait()
                    pltpu.sync_copy(bufb, acc.at[dst_v.at[i + 1]], add=True)

        @pl.when(cid == 0)
        def _():
            edge_loop(sid * RPW0, RPW0)

        @pl.when(cid == 1)
        def _():
            edge_loop(NS * RPW0 + sid * RPW1, RPW1)

        plsc.subcore_barrier()

        @pl.loop(0, RPS, step=B)
        def _(rr):
            pltpu.sync_copy(acc.at[pl.ds(sid * RPS + rr, B)],
                            out_hbm.at[cid, pl.ds(sid * RPS + rr, B)])

    return k(hn, src2d, dst2d)


def _tc_matmul_scale(x, w, deg):
    def body(x_ref, w_ref, d_ref, hn_ref, dv_ref):
        d = d_ref[0] + d_ref[1] + 1.0
        dv = lax.rsqrt(d)
        dv_ref[...] = dv
        h = jnp.dot(x_ref[...], w_ref[...],
                    preferred_element_type=jnp.float32)
        hn_ref[...] = h * dv[:, 0:1]

    return pl.pallas_call(
        body,
        grid=(N // ROW_BLK,),
        in_specs=[pl.BlockSpec((ROW_BLK, D), lambda r: (r, 0)),
                  pl.BlockSpec((D, D), lambda r: (0, 0)),
                  pl.BlockSpec((NC, ROW_BLK, 16), lambda r: (0, r, 0))],
        out_specs=[pl.BlockSpec((ROW_BLK, D), lambda r: (r, 0)),
                   pl.BlockSpec((ROW_BLK, 16), lambda r: (r, 0))],
        out_shape=[jax.ShapeDtypeStruct((N, D), jnp.float32),
                   jax.ShapeDtypeStruct((N, 16), jnp.float32)],
    )(x, w, deg)


def _tc_layer(acc, hnp, dv16, b2d, w):
    def body(a_ref, h_ref, d_ref, b_ref, w_ref, o_ref):
        dv = d_ref[:, 0:1]
        s = (a_ref[0] + a_ref[1] + h_ref[...]) * dv + b_ref[...]
        a = jnp.maximum(s, 0.0)
        o_ref[...] = jnp.dot(a, w_ref[...],
                             preferred_element_type=jnp.float32) * dv

    return pl.pallas_call(
        body,
        grid=(N // ROW_BLK,),
        in_specs=[pl.BlockSpec((NC, ROW_BLK, D), lambda r: (0, r, 0)),
                  pl.BlockSpec((ROW_BLK, D), lambda r: (r, 0)),
                  pl.BlockSpec((ROW_BLK, 16), lambda r: (r, 0)),
                  pl.BlockSpec((1, D), lambda r: (0, 0)),
                  pl.BlockSpec((D, D), lambda r: (0, 0))],
        out_specs=pl.BlockSpec((ROW_BLK, D), lambda r: (r, 0)),
        out_shape=jax.ShapeDtypeStruct((N, D), jnp.float32),
    )(acc, hnp, dv16, b2d, w)


def _tc_final(acc, hnp, dv16, b2d, wl, bl2d):
    def body(a_ref, h_ref, d_ref, b_ref, w_ref, bl_ref, o_ref):
        dv = d_ref[:, 0:1]
        s = (a_ref[0] + a_ref[1] + h_ref[...]) * dv + b_ref[...]
        a = jnp.maximum(s, 0.0)
        logits = jnp.dot(a, w_ref[...],
                         preferred_element_type=jnp.float32) + bl_ref[...]
        m = jnp.max(logits, axis=1, keepdims=True)
        e = jnp.exp(logits - m)
        o_ref[...] = e / jnp.sum(e, axis=1, keepdims=True)

    return pl.pallas_call(
        body,
        grid=(N // ROW_BLK,),
        in_specs=[pl.BlockSpec((NC, ROW_BLK, D), lambda r: (0, r, 0)),
                  pl.BlockSpec((ROW_BLK, D), lambda r: (r, 0)),
                  pl.BlockSpec((ROW_BLK, 16), lambda r: (r, 0)),
                  pl.BlockSpec((1, D), lambda r: (0, 0)),
                  pl.BlockSpec((D, D), lambda r: (0, 0)),
                  pl.BlockSpec((1, D), lambda r: (0, 0))],
        out_specs=pl.BlockSpec((ROW_BLK, D), lambda r: (r, 0)),
        out_shape=jax.ShapeDtypeStruct((N, D), jnp.float32),
    )(acc, hnp, dv16, b2d, wl, bl2d)


def kernel(x, adj_matrix, W1, b1, W2, b2, W3, b3, Wl, bl):
    src = adj_matrix[0].astype(jnp.int32)
    dst = adj_matrix[1].astype(jnp.int32)
    pad = EPAD - E
    src2d = jnp.concatenate(
        [src, jnp.zeros((pad,), jnp.int32)]).reshape(EPAD_ROWS, B)
    dst2d = jnp.concatenate(
        [dst, jnp.full((pad,), N, jnp.int32)]).reshape(EPAD_ROWS, B)

    b1_2d = b1.reshape(1, D)
    b2_2d = b2.reshape(1, D)
    b3_2d = b3.reshape(1, D)
    bl_2d = bl.reshape(1, D)

    deg = _sc_degree(dst2d)
    hn1, dv16 = _tc_matmul_scale(x, W1, deg)

    acc1 = _sc_aggregate(hn1, src2d, dst2d)
    hn2 = _tc_layer(acc1, hn1, dv16, b1_2d, W2)
    acc2 = _sc_aggregate(hn2, src2d, dst2d)
    hn3 = _tc_layer(acc2, hn2, dv16, b2_2d, W3)
    acc3 = _sc_aggregate(hn3, src2d, dst2d)
    return _tc_final(acc3, hn3, dv16, b3_2d, Wl, bl_2d)
